# baseline (speedup 1.0000x reference)
import functools

import jax
import jax.numpy as jnp
from jax import lax
from jax.experimental import pallas as pl
from jax.experimental.pallas import tpu as pltpu
from jax.experimental.pallas import tpu_sc as plsc

VOCAB = 1000000
EMB = 64
HID = 1024
OUT = 128
NTOK = 16384

LANES = 16

CW = 1664
CROWS = 608
CBIG = CROWS * CW
CPT = CBIG // 16
ZCH = CPT // 16

MBLK = 39936
NMAIN = 25
TAIL0 = NMAIN * MBLK
TAILW = CW


def _sc_counts(idx128):
    mesh = plsc.VectorSubcoreMesh(core_axis_name="c", subcore_axis_name="s")

    @functools.partial(
        pl.kernel,
        mesh=mesh,
        out_type=(
            jax.ShapeDtypeStruct((CBIG,), jnp.float32),
            jax.ShapeDtypeStruct((CBIG,), jnp.float32),
        ),
        scratch_types=[
            pltpu.VMEM((4, 128), jnp.int32),
            pltpu.VMEM((128,), jnp.float32),
            pltpu.VMEM((ZCH,), jnp.float32),
            pltpu.VMEM_SHARED((CBIG,), jnp.float32),
            pltpu.SemaphoreType.DMA,
        ],
    )
    def k(idx_hbm, out0_hbm, out1_hbm, idx_v, ones_v, zeros_v, shared, sem):
        s = lax.axis_index("s")
        c = lax.axis_index("c")
        g = s * 2 + c
        pltpu.sync_copy(idx_hbm.at[pl.ds(g * 4, 4)], idx_v)

        zero = jnp.zeros((LANES,), jnp.float32)

        def zv(r, carry):
            zeros_v[pl.ds(pl.multiple_of(r * LANES, LANES), LANES)] = zero
            return carry
        lax.fori_loop(0, ZCH // LANES, zv, 0)
        for l in range(8):
            ones_v[pl.ds(l * LANES, LANES)] = jnp.ones((LANES,), jnp.float32)

        base = pl.multiple_of(s * CPT, 8)
        zcopies = [
            pltpu.async_copy(
                zeros_v, shared.at[pl.ds(base + kk * ZCH, ZCH)], sem
            )
            for kk in range(16)
        ]
        for cp in zcopies:
            cp.wait()
        plsc.subcore_barrier()

        scopies = [
            pltpu.async_copy(ones_v, shared.at[idx_v.at[j]], sem, add=True)
            for j in range(4)
        ]
        for cp in scopies:
            cp.wait()
        plsc.subcore_barrier()

        out = [out0_hbm, out1_hbm]
        for ci in range(2):
            @pl.when(c == ci)
            def _():
                pltpu.sync_copy(
                    shared.at[pl.ds(base, CPT)],
                    out[ci].at[pl.ds(base, CPT)],
                )

    return k(idx128)


def _tc_matvec_main(table_t, c0, c1):
    def body(t_ref, c0_ref, c1_ref, o_ref):
        cc = c0_ref[...] + c1_ref[...]
        tb = t_ref[...]
        acc = jnp.zeros((EMB, 128), jnp.float32)
        for gg in range(MBLK // 128):
            acc = acc + tb[:, gg * 128:(gg + 1) * 128] * cc[gg * 128:(gg + 1) * 128]
        o_ref[...] = acc.reshape(1, EMB, 128)

    return pl.pallas_call(
        body,
        grid=(NMAIN,),
        in_specs=[
            pl.BlockSpec((EMB, MBLK), lambda i: (0, i)),
            pl.BlockSpec((MBLK,), lambda i: (i,)),
            pl.BlockSpec((MBLK,), lambda i: (i,)),
        ],
        out_specs=pl.BlockSpec((1, EMB, 128), lambda i: (i, 0, 0)),
        out_shape=jax.ShapeDtypeStruct((NMAIN, EMB, 128), jnp.float32),
        compiler_params=pltpu.CompilerParams(
            dimension_semantics=("parallel",),
        ),
    )(table_t, c0, c1)


def _tc_ffnn_tail(acc_main, table_t, c0, c1, W1, b1, W2, b2):
    def body(a_ref, t_ref, c0_ref, c1_ref, w1_ref, b1_ref, w2_ref, b2_ref,
             o_ref):
        cc = c0_ref[pl.ds(0, TAILW)] + c1_ref[pl.ds(0, TAILW)]
        tb = t_ref[...]
        acc = jnp.sum(a_ref[...], axis=0)
        for gg in range(TAILW // 128):
            prod = tb[:, gg * 128:(gg + 1) * 128] * cc[gg * 128:(gg + 1) * 128]
            if (gg + 1) * 128 > VOCAB - TAIL0:
                valid = jax.lax.broadcasted_iota(jnp.int32, (1, 128), 1) < (
                    VOCAB - TAIL0 - gg * 128
                )
                prod = jnp.where(valid, prod, 0.0)
            acc = acc + prod

        bagc = jnp.sum(acc, axis=1, keepdims=True) * (1.0 / NTOK)
        h = lax.dot_general(
            bagc, w1_ref[...], (((0,), (0,)), ((), ())),
            preferred_element_type=jnp.float32,
        )
        h = jnp.maximum(h + b1_ref[...], 0.0)
        logits = jnp.dot(h, w2_ref[...], preferred_element_type=jnp.float32)
        logits = logits + b2_ref[...]
        m = jnp.max(logits, axis=-1, keepdims=True)
        lse = jnp.log(jnp.sum(jnp.exp(logits - m), axis=-1, keepdims=True)) + m
        o_ref[...] = logits - lse

    return pl.pallas_call(
        body,
        grid=(1,),
        in_specs=[
            pl.BlockSpec((NMAIN, EMB, 128), lambda i: (0, 0, 0)),
            pl.BlockSpec((EMB, TAILW), lambda i: (0, TAIL0 // TAILW)),
            pl.BlockSpec((13312,), lambda i: (TAIL0 // 13312,)),
            pl.BlockSpec((13312,), lambda i: (TAIL0 // 13312,)),
            pl.BlockSpec((EMB, HID), lambda i: (0, 0)),
            pl.BlockSpec((1, HID), lambda i: (0, 0)),
            pl.BlockSpec((HID, OUT), lambda i: (0, 0)),
            pl.BlockSpec((1, OUT), lambda i: (0, 0)),
        ],
        out_specs=pl.BlockSpec((1, OUT), lambda i: (0, 0)),
        out_shape=jax.ShapeDtypeStruct((1, OUT), jnp.float32),
    )(acc_main, table_t, c0, c1, W1, b1.reshape(1, HID), W2,
      b2.reshape(1, OUT))


def kernel(input, embeddings, W1, b1, W2, b2):
    c0, c1 = _sc_counts(input.reshape(128, 128))
    table_t = embeddings.T
    acc_main = _tc_matvec_main(table_t, c0, c1)
    return _tc_ffnn_tail(acc_main, table_t, c0, c1, W1, b1, W2, b2)


# speedup vs baseline: 1.0008x; 1.0008x over previous
import functools

import jax
import jax.numpy as jnp
from jax import lax
from jax.experimental import pallas as pl
from jax.experimental.pallas import tpu as pltpu
from jax.experimental.pallas import tpu_sc as plsc

VOCAB = 1000000
EMB = 64
HID = 1024
OUT = 128
NTOK = 16384

LANES = 16

CW = 1664
CROWS = 608
CBIG = CROWS * CW
CPT = CBIG // 16
ZCH = CPT // 16

MBLK = 76800
NMAIN = 13
TAIL0 = NMAIN * MBLK
TAILW = CW


def _sc_counts(idx128):
    mesh = plsc.VectorSubcoreMesh(core_axis_name="c", subcore_axis_name="s")

    @functools.partial(
        pl.kernel,
        mesh=mesh,
        out_type=(
            jax.ShapeDtypeStruct((CBIG,), jnp.float32),
            jax.ShapeDtypeStruct((CBIG,), jnp.float32),
        ),
        scratch_types=[
            pltpu.VMEM((4, 128), jnp.int32),
            pltpu.VMEM((128,), jnp.float32),
            pltpu.VMEM((ZCH,), jnp.float32),
            pltpu.VMEM_SHARED((CBIG,), jnp.float32),
            pltpu.SemaphoreType.DMA,
        ],
    )
    def k(idx_hbm, out0_hbm, out1_hbm, idx_v, ones_v, zeros_v, shared, sem):
        s = lax.axis_index("s")
        c = lax.axis_index("c")
        g = s * 2 + c
        pltpu.sync_copy(idx_hbm.at[pl.ds(g * 4, 4)], idx_v)

        zero = jnp.zeros((LANES,), jnp.float32)

        def zv(r, carry):
            zeros_v[pl.ds(pl.multiple_of(r * LANES, LANES), LANES)] = zero
            return carry
        lax.fori_loop(0, ZCH // LANES, zv, 0)
        for l in range(8):
            ones_v[pl.ds(l * LANES, LANES)] = jnp.ones((LANES,), jnp.float32)

        base = pl.multiple_of(s * CPT, 8)
        zcopies = [
            pltpu.async_copy(
                zeros_v, shared.at[pl.ds(base + kk * ZCH, ZCH)], sem
            )
            for kk in range(16)
        ]
        for cp in zcopies:
            cp.wait()
        plsc.subcore_barrier()

        scopies = [
            pltpu.async_copy(ones_v, shared.at[idx_v.at[j]], sem, add=True)
            for j in range(4)
        ]
        for cp in scopies:
            cp.wait()
        plsc.subcore_barrier()

        out = [out0_hbm, out1_hbm]
        for ci in range(2):
            @pl.when(c == ci)
            def _():
                pltpu.sync_copy(
                    shared.at[pl.ds(base, CPT)],
                    out[ci].at[pl.ds(base, CPT)],
                )

    return k(idx128)


def _tc_matvec_main(table_t, c0, c1):
    def body(t_ref, c0_ref, c1_ref, o_ref):
        cc = c0_ref[...] + c1_ref[...]
        tb = t_ref[...]
        acc = jnp.zeros((EMB, 128), jnp.float32)
        for gg in range(MBLK // 128):
            acc = acc + tb[:, gg * 128:(gg + 1) * 128] * cc[gg * 128:(gg + 1) * 128]

        @pl.when(pl.program_id(0) == 0)
        def _():
            o_ref[...] = jnp.zeros_like(o_ref)

        o_ref[...] += acc

    return pl.pallas_call(
        body,
        grid=(NMAIN,),
        in_specs=[
            pl.BlockSpec((EMB, MBLK), lambda i: (0, i)),
            pl.BlockSpec((MBLK,), lambda i: (i,)),
            pl.BlockSpec((MBLK,), lambda i: (i,)),
        ],
        out_specs=pl.BlockSpec((EMB, 128), lambda i: (0, 0)),
        out_shape=jax.ShapeDtypeStruct((EMB, 128), jnp.float32),
    )(table_t, c0, c1)


def _tc_ffnn_tail(acc_main, table_t, c0, c1, W1, b1, W2, b2):
    def body(a_ref, t_ref, c0_ref, c1_ref, w1_ref, b1_ref, w2_ref, b2_ref,
             o_ref):
        cc = c0_ref[pl.ds(0, TAILW)] + c1_ref[pl.ds(0, TAILW)]
        tb = t_ref[...]
        acc = a_ref[...]
        for gg in range(TAILW // 128):
            prod = tb[:, gg * 128:(gg + 1) * 128] * cc[gg * 128:(gg + 1) * 128]
            if (gg + 1) * 128 > VOCAB - TAIL0:
                valid = jax.lax.broadcasted_iota(jnp.int32, (1, 128), 1) < (
                    VOCAB - TAIL0 - gg * 128
                )
                prod = jnp.where(valid, prod, 0.0)
            acc = acc + prod

        bagc = jnp.sum(acc, axis=1, keepdims=True) * (1.0 / NTOK)
        h = lax.dot_general(
            bagc, w1_ref[...], (((0,), (0,)), ((), ())),
            preferred_element_type=jnp.float32,
        )
        h = jnp.maximum(h + b1_ref[...], 0.0)
        logits = jnp.dot(h, w2_ref[...], preferred_element_type=jnp.float32)
        logits = logits + b2_ref[...]
        m = jnp.max(logits, axis=-1, keepdims=True)
        lse = jnp.log(jnp.sum(jnp.exp(logits - m), axis=-1, keepdims=True)) + m
        o_ref[...] = logits - lse

    return pl.pallas_call(
        body,
        grid=(1,),
        in_specs=[
            pl.BlockSpec((EMB, 128), lambda i: (0, 0)),
            pl.BlockSpec((EMB, TAILW), lambda i: (0, TAIL0 // TAILW)),
            pl.BlockSpec((13312,), lambda i: (TAIL0 // 13312,)),
            pl.BlockSpec((13312,), lambda i: (TAIL0 // 13312,)),
            pl.BlockSpec((EMB, HID), lambda i: (0, 0)),
            pl.BlockSpec((1, HID), lambda i: (0, 0)),
            pl.BlockSpec((HID, OUT), lambda i: (0, 0)),
            pl.BlockSpec((1, OUT), lambda i: (0, 0)),
        ],
        out_specs=pl.BlockSpec((1, OUT), lambda i: (0, 0)),
        out_shape=jax.ShapeDtypeStruct((1, OUT), jnp.float32),
    )(acc_main, table_t, c0, c1, W1, b1.reshape(1, HID), W2,
      b2.reshape(1, OUT))


def kernel(input, embeddings, W1, b1, W2, b2):
    c0, c1 = _sc_counts(input.reshape(128, 128))
    table_t = embeddings.T
    acc_main = _tc_matvec_main(table_t, c0, c1)
    return _tc_ffnn_tail(acc_main, table_t, c0, c1, W1, b1, W2, b2)


# speedup vs baseline: 1.0061x; 1.0052x over previous
import functools

import jax
import jax.numpy as jnp
from jax import lax
from jax.experimental import pallas as pl
from jax.experimental.pallas import tpu as pltpu
from jax.experimental.pallas import tpu_sc as plsc

VOCAB = 1000000
EMB = 64
HID = 1024
OUT = 128
NTOK = 16384

LANES = 16

CW = 1664
CROWS = 608
CBIG = CROWS * CW
CPT = CBIG // 16
ZCH = CPT // 16

MBLK = 76800
NMAIN = 13
TAIL0 = NMAIN * MBLK
TAILW = CW


def _sc_counts(idx128):
    mesh = plsc.VectorSubcoreMesh(core_axis_name="c", subcore_axis_name="s")

    @functools.partial(
        pl.kernel,
        mesh=mesh,
        out_type=(
            jax.ShapeDtypeStruct((CBIG,), jnp.float32),
            jax.ShapeDtypeStruct((CBIG,), jnp.float32),
        ),
        scratch_types=[
            pltpu.VMEM((4, 128), jnp.int32),
            pltpu.VMEM((128,), jnp.float32),
            pltpu.VMEM((ZCH,), jnp.float32),
            pltpu.VMEM_SHARED((CBIG,), jnp.float32),
            pltpu.SemaphoreType.DMA,
        ],
    )
    def k(idx_hbm, out0_hbm, out1_hbm, idx_v, ones_v, zeros_v, shared, sem):
        s = lax.axis_index("s")
        c = lax.axis_index("c")
        g = s * 2 + c
        pltpu.sync_copy(idx_hbm.at[pl.ds(g * 4, 4)], idx_v)

        zero = jnp.zeros((LANES,), jnp.float32)

        def zv(r, carry):
            zeros_v[pl.ds(pl.multiple_of(r * LANES, LANES), LANES)] = zero
            return carry
        lax.fori_loop(0, ZCH // LANES, zv, 0)
        for l in range(8):
            ones_v[pl.ds(l * LANES, LANES)] = jnp.ones((LANES,), jnp.float32)

        base = pl.multiple_of(s * CPT, 8)
        zcopies = [
            pltpu.async_copy(
                zeros_v, shared.at[pl.ds(base + kk * ZCH, ZCH)], sem
            )
            for kk in range(16)
        ]
        for cp in zcopies:
            cp.wait()
        plsc.subcore_barrier()

        scopies = [
            pltpu.async_copy(ones_v, shared.at[idx_v.at[j]], sem, add=True)
            for j in range(4)
        ]
        for cp in scopies:
            cp.wait()
        plsc.subcore_barrier()

        out = [out0_hbm, out1_hbm]
        for ci in range(2):
            @pl.when(c == ci)
            def _():
                pltpu.sync_copy(
                    shared.at[pl.ds(base, CPT)],
                    out[ci].at[pl.ds(base, CPT)],
                )

    return k(idx128)


NSTR = 3
SBLK = 13312
GMAIN = 25


def _tc_matvec_main(table_t, c0, c1):
    def body(t0, t1, t2, a0, a1, a2, b0, b1, b2, o_ref):
        acc = jnp.zeros((EMB, 128), jnp.float32)
        for t_ref, ca_ref, cb_ref in ((t0, a0, b0), (t1, a1, b1),
                                      (t2, a2, b2)):
            cc = ca_ref[...] + cb_ref[...]
            tb = t_ref[...]
            for gg in range(SBLK // 128):
                acc = acc + (tb[:, gg * 128:(gg + 1) * 128]
                             * cc[gg * 128:(gg + 1) * 128])

        @pl.when(pl.program_id(0) == 0)
        def _():
            o_ref[...] = jnp.zeros_like(o_ref)

        o_ref[...] += acc

    tspecs = [
        pl.BlockSpec((EMB, SBLK), lambda i, j=j: (0, NSTR * i + j))
        for j in range(NSTR)
    ]
    cspecs = [
        pl.BlockSpec((SBLK,), lambda i, j=j: (NSTR * i + j,))
        for j in range(NSTR)
    ]
    return pl.pallas_call(
        body,
        grid=(GMAIN,),
        in_specs=tspecs + cspecs + cspecs,
        out_specs=pl.BlockSpec((EMB, 128), lambda i: (0, 0)),
        out_shape=jax.ShapeDtypeStruct((EMB, 128), jnp.float32),
    )(table_t, table_t, table_t, c0, c0, c0, c1, c1, c1)


def _tc_ffnn_tail(acc_main, table_t, c0, c1, W1, b1, W2, b2):
    def body(a_ref, t_ref, c0_ref, c1_ref, w1_ref, b1_ref, w2_ref, b2_ref,
             o_ref):
        cc = c0_ref[pl.ds(0, TAILW)] + c1_ref[pl.ds(0, TAILW)]
        tb = t_ref[...]
        acc = a_ref[...]
        for gg in range(TAILW // 128):
            prod = tb[:, gg * 128:(gg + 1) * 128] * cc[gg * 128:(gg + 1) * 128]
            if (gg + 1) * 128 > VOCAB - TAIL0:
                valid = jax.lax.broadcasted_iota(jnp.int32, (1, 128), 1) < (
                    VOCAB - TAIL0 - gg * 128
                )
                prod = jnp.where(valid, prod, 0.0)
            acc = acc + prod

        bagc = jnp.sum(acc, axis=1, keepdims=True) * (1.0 / NTOK)
        h = lax.dot_general(
            bagc, w1_ref[...], (((0,), (0,)), ((), ())),
            preferred_element_type=jnp.float32,
        )
        h = jnp.maximum(h + b1_ref[...], 0.0)
        logits = jnp.dot(h, w2_ref[...], preferred_element_type=jnp.float32)
        logits = logits + b2_ref[...]
        m = jnp.max(logits, axis=-1, keepdims=True)
        lse = jnp.log(jnp.sum(jnp.exp(logits - m), axis=-1, keepdims=True)) + m
        o_ref[...] = logits - lse

    return pl.pallas_call(
        body,
        grid=(1,),
        in_specs=[
            pl.BlockSpec((EMB, 128), lambda i: (0, 0)),
            pl.BlockSpec((EMB, TAILW), lambda i: (0, TAIL0 // TAILW)),
            pl.BlockSpec((13312,), lambda i: (TAIL0 // 13312,)),
            pl.BlockSpec((13312,), lambda i: (TAIL0 // 13312,)),
            pl.BlockSpec((EMB, HID), lambda i: (0, 0)),
            pl.BlockSpec((1, HID), lambda i: (0, 0)),
            pl.BlockSpec((HID, OUT), lambda i: (0, 0)),
            pl.BlockSpec((1, OUT), lambda i: (0, 0)),
        ],
        out_specs=pl.BlockSpec((1, OUT), lambda i: (0, 0)),
        out_shape=jax.ShapeDtypeStruct((1, OUT), jnp.float32),
    )(acc_main, table_t, c0, c1, W1, b1.reshape(1, HID), W2,
      b2.reshape(1, OUT))


def kernel(input, embeddings, W1, b1, W2, b2):
    c0, c1 = _sc_counts(input.reshape(128, 128))
    table_t = embeddings.T
    acc_main = _tc_matvec_main(table_t, c0, c1)
    return _tc_ffnn_tail(acc_main, table_t, c0, c1, W1, b1, W2, b2)
